# Initial kernel scaffold; baseline (speedup 1.0000x reference)
import jax
import jax.numpy as jnp
from jax.experimental import pallas as pl


def kernel(input, codebook):
    raise NotImplementedError("write your pallas kernel here")


# speedup vs baseline: 2.6522x; 2.6522x over previous
import jax
import jax.numpy as jnp
from jax import lax
from jax.experimental import pallas as pl

_N, _D, _K = 65536, 256, 1024
_BN = 512
_NB = _N // _BN


def _tc_body(x_ref, cb_ref, sim_ref, lab_ref, pred_ref):
    x = x_ref[...]
    cb = cb_ref[...]
    sim = lax.dot_general(x, cb, (((1,), (1,)), ((), ())),
                          preferred_element_type=jnp.float32)
    sim_ref[...] = sim
    lab = jnp.argmax(sim, axis=1).astype(jnp.int32)
    lab_ref[0, 0, :] = lab
    onehot = (lax.broadcasted_iota(jnp.int32, (_BN, _K), 1)
              == lab[:, None]).astype(jnp.float32)
    pred_ref[...] = lax.dot_general(onehot, cb, (((1,), (0,)), ((), ())),
                                    preferred_element_type=jnp.float32)


def kernel(input, codebook):
    sim, lab3, preds = pl.pallas_call(
        _tc_body,
        grid=(_NB,),
        in_specs=[pl.BlockSpec((_BN, _D), lambda i: (i, 0)),
                  pl.BlockSpec((_K, _D), lambda i: (0, 0))],
        out_specs=[pl.BlockSpec((_BN, _K), lambda i: (i, 0)),
                   pl.BlockSpec((1, 1, _BN), lambda i: (i, 0, 0)),
                   pl.BlockSpec((_BN, _D), lambda i: (i, 0))],
        out_shape=[jax.ShapeDtypeStruct((_N, _K), jnp.float32),
                   jax.ShapeDtypeStruct((_NB, 1, _BN), jnp.int32),
                   jax.ShapeDtypeStruct((_N, _D), jnp.float32)],
    )(input, codebook)
    labels = lab3.reshape(_N).astype(jnp.int64)
    return (preds, labels, sim)
